# baseline (speedup 1.0000x reference)
import jax
import jax.numpy as jnp
from jax import lax
from jax.experimental import pallas as pl
from jax.experimental.pallas import tpu as pltpu
from jax.experimental.pallas import tpu_sc as plsc

B = 16384
D = 64
H = 128
PK = 8192

_info = plsc.get_sparse_core_info()
_NC, _NS = _info.num_cores, _info.num_subcores
_NW = _NC * _NS
_BPW = B // _NW


def _vh(V):
    half_blocks = (V // 2 + PK - 1) // PK
    return half_blocks * PK


def _pack_body(in1_ref, in2_ref, o_ref):
    eye = (lax.broadcasted_iota(jnp.int32, (D, D), 0)
           == lax.broadcasted_iota(jnp.int32, (D, D), 1)).astype(jnp.float32)
    o_ref[:, :D] = lax.dot_general(
        in1_ref[...], eye, (((0,), (0,)), ((), ())),
        preferred_element_type=jnp.float32)
    o_ref[:, D:] = lax.dot_general(
        in2_ref[...], eye, (((0,), (0,)), ((), ())),
        preferred_element_type=jnp.float32)


def _pack(tabT):
    V = tabT.shape[1]
    Vh = _vh(V)
    nblk = Vh // PK
    last = (V + PK - 1) // PK - 1
    return pl.pallas_call(
        _pack_body,
        grid=(nblk,),
        in_specs=[
            pl.BlockSpec((D, PK), lambda i: (0, i)),
            pl.BlockSpec(
                (D, PK), lambda i, n=nblk, m=last: (0, jnp.minimum(i + n, m))),
        ],
        out_specs=pl.BlockSpec((PK, 2 * D), lambda i: (i, 0)),
        out_shape=jax.ShapeDtypeStruct((Vh, 2 * D), jnp.float32),
    )(tabT, tabT)


_GCH = 128


def _gather_one_table(idx_hbm, pk_hbm, out_hbm, base, Vh,
                      idx_v, row_v, buf_v, sem):
    pltpu.sync_copy(idx_hbm.at[pl.ds(base, _BPW)], idx_v)
    for i in range(_BPW // 16):
        vec = idx_v[pl.ds(i * 16, 16)]
        row_v[pl.ds(i * 16, 16)] = jnp.where(vec >= Vh, vec - Vh, vec)
    for c in range(_BPW // _GCH):
        pltpu.async_copy(
            pk_hbm.at[row_v.at[pl.ds(c * _GCH, _GCH)]],
            buf_v.at[pl.ds(c * _GCH, _GCH)], sem)
    pltpu.make_async_copy(
        pk_hbm.at[pl.ds(0, _BPW)], buf_v, sem).wait()
    pltpu.sync_copy(buf_v, out_hbm.at[pl.ds(base, _BPW)])


def _gather_body(uid, rid, cid, vid, upk, rpk, cpk, vpk,
                 out_u, out_r, out_c, out_v, idx_v, row_v, buf_v, sem):
    wid = lax.axis_index("s") * _NC + lax.axis_index("c")
    base = wid * _BPW
    for idx_hbm, pk_hbm, out_hbm, Vh in (
        (uid, upk, out_u, _vh(1000000)), (rid, rpk, out_r, _vh(1000)),
        (cid, cpk, out_c, _vh(100000)), (vid, vpk, out_v, _vh(1000000))):
        _gather_one_table(idx_hbm, pk_hbm, out_hbm, base, Vh,
                          idx_v, row_v, buf_v, sem)


_feat2_sds = jax.ShapeDtypeStruct((B, 2 * D), jnp.float32)

_gather = pl.kernel(
    _gather_body,
    out_type=(_feat2_sds, _feat2_sds, _feat2_sds, _feat2_sds),
    mesh=plsc.VectorSubcoreMesh(core_axis_name="c", subcore_axis_name="s"),
    scratch_types=[
        pltpu.VMEM((_BPW,), jnp.int32),
        pltpu.VMEM((_BPW,), jnp.int32),
        pltpu.VMEM((_BPW, 2 * D), jnp.float32),
        pltpu.SemaphoreType.DMA,
    ],
)


def _sel(x2_ref, idx_col, Vh):
    x2 = x2_ref[...]
    hi = (idx_col >= Vh).astype(jnp.float32)
    return x2[:, :D] * (1.0 - hi) + x2[:, D:] * hi


def _mlp_body(u_ref, r_ref, c_ref, v_ref, ui_ref, ri_ref, ci_ref, vi_ref,
              w1_ref, b1_ref, w2_ref, b2_ref, o_ref):
    x = jnp.concatenate(
        [_sel(u_ref, ui_ref[...], _vh(1000000)),
         _sel(r_ref, ri_ref[...], _vh(1000)),
         _sel(c_ref, ci_ref[...], _vh(100000)),
         _sel(v_ref, vi_ref[...], _vh(1000000))],
        axis=-1)
    h = jnp.dot(x, w1_ref[...], preferred_element_type=jnp.float32)
    h = jnp.maximum(h + b1_ref[...], 0.0)
    o = jnp.dot(h, w2_ref[...], preferred_element_type=jnp.float32)
    o_ref[...] = o + b2_ref[...]


def _mlp(u2, r2, c2, v2, ui, ri, ci, vi, W1, b1, W2, b2):
    blk = 4096
    grid = (B // blk,)
    feat_spec = pl.BlockSpec((blk, 2 * D), lambda i: (i, 0))
    idx_spec = pl.BlockSpec((blk, 1), lambda i: (i, 0))
    return pl.pallas_call(
        _mlp_body,
        grid=grid,
        in_specs=[
            feat_spec, feat_spec, feat_spec, feat_spec,
            idx_spec, idx_spec, idx_spec, idx_spec,
            pl.BlockSpec((4 * D, H), lambda i: (0, 0)),
            pl.BlockSpec((1, H), lambda i: (0, 0)),
            pl.BlockSpec((H, D), lambda i: (0, 0)),
            pl.BlockSpec((1, D), lambda i: (0, 0)),
        ],
        out_specs=pl.BlockSpec((blk, D), lambda i: (i, 0)),
        out_shape=jax.ShapeDtypeStruct((B, D), jnp.float32),
    )(u2, r2, c2, v2,
      ui.reshape(B, 1), ri.reshape(B, 1), ci.reshape(B, 1), vi.reshape(B, 1),
      W1, b1.reshape(1, H), W2, b2.reshape(1, D))


def kernel(user_id, region, city, item_id_currentview,
           user_table, region_table, city_table, view_table,
           W1, b1, W2, b2):
    upk = _pack(user_table.T)
    rpk = _pack(region_table.T)
    cpk = _pack(city_table.T)
    vpk = _pack(view_table.T)
    u2, r2, c2, v2 = _gather(user_id, region, city, item_id_currentview,
                             upk, rpk, cpk, vpk)
    return _mlp(u2, r2, c2, v2,
                user_id, region, city, item_id_currentview, W1, b1, W2, b2)


# speedup vs baseline: 1.0289x; 1.0289x over previous
import jax
import jax.numpy as jnp
from jax import lax
from jax.experimental import pallas as pl
from jax.experimental.pallas import tpu as pltpu
from jax.experimental.pallas import tpu_sc as plsc

B = 16384
D = 64
H = 128
PK = 16384

_info = plsc.get_sparse_core_info()
_NC, _NS = _info.num_cores, _info.num_subcores
_NW = _NC * _NS
_BPW = B // _NW


def _vh(V):
    half_blocks = (V // 2 + PK - 1) // PK
    return half_blocks * PK


def _pack_body(in1_ref, in2_ref, o_ref):
    eye = (lax.broadcasted_iota(jnp.int32, (D, D), 0)
           == lax.broadcasted_iota(jnp.int32, (D, D), 1)).astype(jnp.float32)
    o_ref[:, :D] = lax.dot_general(
        in1_ref[...], eye, (((0,), (0,)), ((), ())),
        preferred_element_type=jnp.float32)
    o_ref[:, D:] = lax.dot_general(
        in2_ref[...], eye, (((0,), (0,)), ((), ())),
        preferred_element_type=jnp.float32)


def _pack(tabT):
    V = tabT.shape[1]
    Vh = _vh(V)
    nblk = Vh // PK
    last = (V + PK - 1) // PK - 1
    return pl.pallas_call(
        _pack_body,
        grid=(nblk,),
        in_specs=[
            pl.BlockSpec((D, PK), lambda i: (0, i)),
            pl.BlockSpec(
                (D, PK), lambda i, n=nblk, m=last: (0, jnp.minimum(i + n, m))),
        ],
        out_specs=pl.BlockSpec((PK, 2 * D), lambda i: (i, 0)),
        out_shape=jax.ShapeDtypeStruct((Vh, 2 * D), jnp.float32),
    )(tabT, tabT)


_GCH = 128


def _gather_one_table(idx_hbm, pk_hbm, out_hbm, base, Vh,
                      idx_v, row_v, buf_v, sem):
    pltpu.sync_copy(idx_hbm.at[pl.ds(base, _BPW)], idx_v)
    for i in range(_BPW // 16):
        vec = idx_v[pl.ds(i * 16, 16)]
        row_v[pl.ds(i * 16, 16)] = jnp.where(vec >= Vh, vec - Vh, vec)
    for c in range(_BPW // _GCH):
        pltpu.async_copy(
            pk_hbm.at[row_v.at[pl.ds(c * _GCH, _GCH)]],
            buf_v.at[pl.ds(c * _GCH, _GCH)], sem)
    pltpu.make_async_copy(
        pk_hbm.at[pl.ds(0, _BPW)], buf_v, sem).wait()
    pltpu.sync_copy(buf_v, out_hbm.at[pl.ds(base, _BPW)])


def _gather_body(uid, rid, cid, vid, upk, rpk, cpk, vpk,
                 out_u, out_r, out_c, out_v, idx_v, row_v, buf_v, sem):
    wid = lax.axis_index("s") * _NC + lax.axis_index("c")
    base = wid * _BPW
    for idx_hbm, pk_hbm, out_hbm, Vh in (
        (uid, upk, out_u, _vh(1000000)), (rid, rpk, out_r, _vh(1000)),
        (cid, cpk, out_c, _vh(100000)), (vid, vpk, out_v, _vh(1000000))):
        _gather_one_table(idx_hbm, pk_hbm, out_hbm, base, Vh,
                          idx_v, row_v, buf_v, sem)


_feat2_sds = jax.ShapeDtypeStruct((B, 2 * D), jnp.float32)

_gather = pl.kernel(
    _gather_body,
    out_type=(_feat2_sds, _feat2_sds, _feat2_sds, _feat2_sds),
    mesh=plsc.VectorSubcoreMesh(core_axis_name="c", subcore_axis_name="s"),
    scratch_types=[
        pltpu.VMEM((_BPW,), jnp.int32),
        pltpu.VMEM((_BPW,), jnp.int32),
        pltpu.VMEM((_BPW, 2 * D), jnp.float32),
        pltpu.SemaphoreType.DMA,
    ],
)


def _sel(x2_ref, idx_col, Vh):
    x2 = x2_ref[...]
    hi = (idx_col >= Vh).astype(jnp.float32)
    return x2[:, :D] * (1.0 - hi) + x2[:, D:] * hi


def _mlp_body(u_ref, r_ref, c_ref, v_ref, ui_ref, ri_ref, ci_ref, vi_ref,
              w1_ref, b1_ref, w2_ref, b2_ref, o_ref):
    x = jnp.concatenate(
        [_sel(u_ref, ui_ref[...], _vh(1000000)),
         _sel(r_ref, ri_ref[...], _vh(1000)),
         _sel(c_ref, ci_ref[...], _vh(100000)),
         _sel(v_ref, vi_ref[...], _vh(1000000))],
        axis=-1)
    h = jnp.dot(x, w1_ref[...], preferred_element_type=jnp.float32)
    h = jnp.maximum(h + b1_ref[...], 0.0)
    o = jnp.dot(h, w2_ref[...], preferred_element_type=jnp.float32)
    o_ref[...] = o + b2_ref[...]


def _mlp(u2, r2, c2, v2, ui, ri, ci, vi, W1, b1, W2, b2):
    blk = 4096
    grid = (B // blk,)
    feat_spec = pl.BlockSpec((blk, 2 * D), lambda i: (i, 0))
    idx_spec = pl.BlockSpec((blk, 1), lambda i: (i, 0))
    return pl.pallas_call(
        _mlp_body,
        grid=grid,
        in_specs=[
            feat_spec, feat_spec, feat_spec, feat_spec,
            idx_spec, idx_spec, idx_spec, idx_spec,
            pl.BlockSpec((4 * D, H), lambda i: (0, 0)),
            pl.BlockSpec((1, H), lambda i: (0, 0)),
            pl.BlockSpec((H, D), lambda i: (0, 0)),
            pl.BlockSpec((1, D), lambda i: (0, 0)),
        ],
        out_specs=pl.BlockSpec((blk, D), lambda i: (i, 0)),
        out_shape=jax.ShapeDtypeStruct((B, D), jnp.float32),
    )(u2, r2, c2, v2,
      ui.reshape(B, 1), ri.reshape(B, 1), ci.reshape(B, 1), vi.reshape(B, 1),
      W1, b1.reshape(1, H), W2, b2.reshape(1, D))


def kernel(user_id, region, city, item_id_currentview,
           user_table, region_table, city_table, view_table,
           W1, b1, W2, b2):
    upk = _pack(user_table.T)
    rpk = _pack(region_table.T)
    cpk = _pack(city_table.T)
    vpk = _pack(view_table.T)
    u2, r2, c2, v2 = _gather(user_id, region, city, item_id_currentview,
                             upk, rpk, cpk, vpk)
    return _mlp(u2, r2, c2, v2,
                user_id, region, city, item_id_currentview, W1, b1, W2, b2)


# speedup vs baseline: 1.0648x; 1.0349x over previous
import jax
import jax.numpy as jnp
from jax import lax
from jax.experimental import pallas as pl
from jax.experimental.pallas import tpu as pltpu
from jax.experimental.pallas import tpu_sc as plsc

B = 16384
D = 64
H = 128
PK = 16384

_info = plsc.get_sparse_core_info()
_NC, _NS = _info.num_cores, _info.num_subcores
_NW = _NC * _NS
_BPW = B // _NW


def _vh(V):
    half_blocks = (V // 2 + PK - 1) // PK
    return half_blocks * PK


def _pack_body(in1_ref, in2_ref, o_ref):
    eye = (lax.broadcasted_iota(jnp.int32, (D, D), 0)
           == lax.broadcasted_iota(jnp.int32, (D, D), 1)).astype(jnp.float32)
    o_ref[:, :D] = lax.dot_general(
        in1_ref[...], eye, (((0,), (0,)), ((), ())),
        preferred_element_type=jnp.float32)
    o_ref[:, D:] = lax.dot_general(
        in2_ref[...], eye, (((0,), (0,)), ((), ())),
        preferred_element_type=jnp.float32)


def _pack(tabT):
    V = tabT.shape[1]
    Vh = _vh(V)
    nblk = Vh // PK
    last = (V + PK - 1) // PK - 1
    return pl.pallas_call(
        _pack_body,
        grid=(nblk,),
        in_specs=[
            pl.BlockSpec((D, PK), lambda i: (0, i)),
            pl.BlockSpec(
                (D, PK), lambda i, n=nblk, m=last: (0, jnp.minimum(i + n, m))),
        ],
        out_specs=pl.BlockSpec((PK, 2 * D), lambda i: (i, 0)),
        out_shape=jax.ShapeDtypeStruct((Vh, 2 * D), jnp.float32),
    )(tabT, tabT)


_GCH = 128


def _gather_one_table(idx_hbm, pk_hbm, out_hbm, base, Vh,
                      idx_v, row_v, buf_v, sem):
    pltpu.sync_copy(idx_hbm.at[pl.ds(base, _BPW)], idx_v)
    for i in range(_BPW // 16):
        vec = idx_v[pl.ds(i * 16, 16)]
        row_v[pl.ds(i * 16, 16)] = jnp.where(vec >= Vh, vec - Vh, vec)
    for c in range(_BPW // _GCH):
        pltpu.async_copy(
            pk_hbm.at[row_v.at[pl.ds(c * _GCH, _GCH)]],
            buf_v.at[pl.ds(c * _GCH, _GCH)], sem)
    pltpu.make_async_copy(
        pk_hbm.at[pl.ds(0, _BPW)], buf_v, sem).wait()
    pltpu.sync_copy(buf_v, out_hbm.at[pl.ds(base, _BPW)])


_feat2_sds = jax.ShapeDtypeStruct((B, 2 * D), jnp.float32)


def _make_gather(V):
    Vh = _vh(V)

    def body(idx_hbm, pk_hbm, out_hbm, idx_v, row_v, buf_v, sem):
        wid = lax.axis_index("s") * _NC + lax.axis_index("c")
        base = wid * _BPW
        _gather_one_table(idx_hbm, pk_hbm, out_hbm, base, Vh,
                          idx_v, row_v, buf_v, sem)

    return pl.kernel(
        body,
        out_type=_feat2_sds,
        mesh=plsc.VectorSubcoreMesh(core_axis_name="c", subcore_axis_name="s"),
        scratch_types=[
            pltpu.VMEM((_BPW,), jnp.int32),
            pltpu.VMEM((_BPW,), jnp.int32),
            pltpu.VMEM((_BPW, 2 * D), jnp.float32),
            pltpu.SemaphoreType.DMA,
        ],
    )


_gather_u = _make_gather(1000000)
_gather_r = _make_gather(1000)
_gather_c = _make_gather(100000)
_gather_v = _make_gather(1000000)


def _sel(x2_ref, idx_col, Vh):
    x2 = x2_ref[...]
    hi = (idx_col >= Vh).astype(jnp.float32)
    return x2[:, :D] * (1.0 - hi) + x2[:, D:] * hi


def _mlp_body(u_ref, r_ref, c_ref, v_ref, ui_ref, ri_ref, ci_ref, vi_ref,
              w1_ref, b1_ref, w2_ref, b2_ref, o_ref):
    x = jnp.concatenate(
        [_sel(u_ref, ui_ref[...], _vh(1000000)),
         _sel(r_ref, ri_ref[...], _vh(1000)),
         _sel(c_ref, ci_ref[...], _vh(100000)),
         _sel(v_ref, vi_ref[...], _vh(1000000))],
        axis=-1)
    h = jnp.dot(x, w1_ref[...], preferred_element_type=jnp.float32)
    h = jnp.maximum(h + b1_ref[...], 0.0)
    o = jnp.dot(h, w2_ref[...], preferred_element_type=jnp.float32)
    o_ref[...] = o + b2_ref[...]


def _mlp(u2, r2, c2, v2, ui, ri, ci, vi, W1, b1, W2, b2):
    blk = 4096
    grid = (B // blk,)
    feat_spec = pl.BlockSpec((blk, 2 * D), lambda i: (i, 0))
    idx_spec = pl.BlockSpec((blk, 1), lambda i: (i, 0))
    return pl.pallas_call(
        _mlp_body,
        grid=grid,
        in_specs=[
            feat_spec, feat_spec, feat_spec, feat_spec,
            idx_spec, idx_spec, idx_spec, idx_spec,
            pl.BlockSpec((4 * D, H), lambda i: (0, 0)),
            pl.BlockSpec((1, H), lambda i: (0, 0)),
            pl.BlockSpec((H, D), lambda i: (0, 0)),
            pl.BlockSpec((1, D), lambda i: (0, 0)),
        ],
        out_specs=pl.BlockSpec((blk, D), lambda i: (i, 0)),
        out_shape=jax.ShapeDtypeStruct((B, D), jnp.float32),
    )(u2, r2, c2, v2,
      ui.reshape(B, 1), ri.reshape(B, 1), ci.reshape(B, 1), vi.reshape(B, 1),
      W1, b1.reshape(1, H), W2, b2.reshape(1, D))


def kernel(user_id, region, city, item_id_currentview,
           user_table, region_table, city_table, view_table,
           W1, b1, W2, b2):
    upk = _pack(user_table.T)
    u2 = _gather_u(user_id, upk)
    vpk = _pack(view_table.T)
    v2 = _gather_v(item_id_currentview, vpk)
    cpk = _pack(city_table.T)
    c2 = _gather_c(city, cpk)
    rpk = _pack(region_table.T)
    r2 = _gather_r(region, rpk)
    return _mlp(u2, r2, c2, v2,
                user_id, region, city, item_id_currentview, W1, b1, W2, b2)
